# baseline (speedup 1.0000x reference)
import functools

import jax
import jax.numpy as jnp
from jax import lax
from jax.experimental import pallas as pl
from jax.experimental.pallas import tpu as pltpu
from jax.experimental.pallas import tpu_sc as plsc

_NC = 2
_NS = 16
_LANES = 16
_NW = _NC * _NS


def _make_prep(B, S, H, Lmax, N):
    bpw = B // _NW
    ngrp = bpw // _LANES
    mesh = plsc.VectorSubcoreMesh(
        core_axis_name="c", subcore_axis_name="s",
        num_cores=_NC, num_subcores=_NS)

    @functools.partial(
        pl.kernel,
        mesh=mesh,
        compiler_params=pltpu.CompilerParams(needs_layout_passes=False),
        out_type=(
            jax.ShapeDtypeStruct((B, H), jnp.float32),
            jax.ShapeDtypeStruct((B, Lmax), jnp.int32),
            jax.ShapeDtypeStruct((B,), jnp.int32),
        ),
        scratch_types=(
            pltpu.VMEM((bpw,), jnp.int32),
            pltpu.VMEM((S * Lmax,), jnp.int32),
            pltpu.VMEM((S,), jnp.int32),
            pltpu.VMEM((bpw,), jnp.int32),
            pltpu.VMEM((bpw, H), jnp.float32),
            pltpu.VMEM((bpw, Lmax), jnp.int32),
            pltpu.VMEM((bpw,), jnp.int32),
            pltpu.SemaphoreType.DMA,
        ),
    )
    def prep(sub_flat_hbm, choice_hbm, etab_hbm, ltab_hbm,
             vsub_hbm, nidx_hbm, lens_hbm,
             svec, etab, ltab, gidx, rows, nidx, lens, sem):
        wid = lax.axis_index("s") * _NC + lax.axis_index("c")
        base = wid * bpw
        pltpu.sync_copy(choice_hbm.at[pl.ds(base, bpw)], svec)
        pltpu.sync_copy(etab_hbm, etab)
        pltpu.sync_copy(ltab_hbm, ltab)
        lane = lax.iota(jnp.int32, _LANES)
        for g in range(ngrp):
            s16 = svec[pl.ds(g * _LANES, _LANES)]
            bvec = lane + (base + g * _LANES)
            gidx[pl.ds(g * _LANES, _LANES)] = bvec * S + s16
            lens[pl.ds(g * _LANES, _LANES)] = plsc.load_gather(ltab, [s16])
        pltpu.async_copy(sub_flat_hbm.at[gidx], rows, sem).wait()
        pltpu.sync_copy(rows, vsub_hbm.at[pl.ds(base, bpw)])
        pltpu.sync_copy(lens, lens_hbm.at[pl.ds(base, bpw)])
        for g in range(ngrp):
            s16 = svec[pl.ds(g * _LANES, _LANES)]
            for u in range(_LANES):
                b = g * _LANES + u
                cvec = plsc.load_gather(etab, [s16[u] * Lmax + lane])
                nidx[b, :] = cvec + (base + b) * N
        pltpu.sync_copy(nidx, nidx_hbm.at[pl.ds(base, bpw)])

    return prep


def _proj_body(vsub_ref, wq_ref, wk_ref, out_ref):
    m = lax.dot_general(
        wq_ref[...], wk_ref[...], (((1,), (1,)), ((), ())),
        preferred_element_type=jnp.float32)
    out_ref[...] = jnp.dot(vsub_ref[...], m, preferred_element_type=jnp.float32)


def _make_dots(B, N, H, Lmax):
    bpw = B // _NW
    inv_sqrt_h = 1.0 / (H ** 0.5)
    unroll = _LANES
    mesh = plsc.VectorSubcoreMesh(
        core_axis_name="c", subcore_axis_name="s",
        num_cores=_NC, num_subcores=_NS)

    @functools.partial(
        pl.kernel,
        mesh=mesh,
        compiler_params=pltpu.CompilerParams(needs_layout_passes=False),
        out_type=jax.ShapeDtypeStruct((B, Lmax), jnp.float32),
        scratch_types=(
            pltpu.VMEM((bpw, H), jnp.float32),
            pltpu.VMEM((bpw, Lmax), jnp.int32),
            pltpu.VMEM((bpw,), jnp.int32),
            pltpu.VMEM((Lmax, H), jnp.float32),
            pltpu.VMEM((Lmax, H), jnp.float32),
            pltpu.VMEM((bpw, Lmax), jnp.float32),
            pltpu.SemaphoreType.DMA,
            pltpu.SemaphoreType.DMA,
        ),
    )
    def dots(node_hbm, v_hbm, nidx_hbm, lens_hbm, out_hbm,
             vbuf, idxbuf, lenbuf, rows0, rows1, outbuf, sem0, sem1):
        wid = lax.axis_index("s") * _NC + lax.axis_index("c")
        base = wid * bpw
        pltpu.sync_copy(nidx_hbm.at[pl.ds(base, bpw)], idxbuf)
        pltpu.sync_copy(v_hbm.at[pl.ds(base, bpw)], vbuf)
        pltpu.sync_copy(lens_hbm.at[pl.ds(base, bpw)], lenbuf)
        lane = lax.iota(jnp.int32, _LANES)
        bufs = (rows0, rows1)
        sems = (sem0, sem1)
        cps = [None, None]
        cps[0] = pltpu.async_copy(node_hbm.at[idxbuf.at[0]], rows0, sem0)
        for b in range(bpw):
            if b + 1 < bpw:
                nxt = (b + 1) % 2
                cps[nxt] = pltpu.async_copy(
                    node_hbm.at[idxbuf.at[b + 1]], bufs[nxt], sems[nxt])
            cps[b % 2].wait()
            rows = bufs[b % 2]
            ones = jnp.full((_LANES,), 1, jnp.int32)

            def jbody(i, acc, rows=rows, b=b, ones=ones):
                vvec = vbuf[b, pl.ds(i * unroll, unroll)]
                for u in range(unroll):
                    col = plsc.load_gather(
                        rows, [lane, ones * (i * unroll + u)])
                    acc = acc + col * vvec[u]
                return acc

            acc = lax.fori_loop(
                0, H // unroll, jbody, jnp.zeros((_LANES,), jnp.float32))
            score = acc * inv_sqrt_h
            sig = 1.0 / (1.0 + jnp.exp(-score))
            lvec = lenbuf[pl.ds((b // _LANES) * _LANES, _LANES)]
            outbuf[b, :] = jnp.where(lane < lvec[b % _LANES], sig, 0.0)
        pltpu.sync_copy(outbuf, out_hbm.at[pl.ds(base, bpw)])

    return dots


def kernel(node_embeddings, substation_embeddings, Wq, Wk,
           sub_choice, sub_elem_idx, sub_elem_len):
    B, N, H = node_embeddings.shape
    S, Lmax = sub_elem_idx.shape
    sub_flat = substation_embeddings.reshape(B * S, H)
    node_flat = node_embeddings.reshape(B * N, H)

    prep = _make_prep(B, S, H, Lmax, N)
    vsub, nidx, lens = prep(
        sub_flat, sub_choice[:, 0], sub_elem_idx.reshape(-1), sub_elem_len)

    v = pl.pallas_call(
        _proj_body,
        out_shape=jax.ShapeDtypeStruct((B, H), jnp.float32),
    )(vsub, Wq, Wk)

    dots = _make_dots(B, N, H, Lmax)
    logits = dots(node_flat, v, nidx, lens)
    return (logits[:, None, :], sub_choice)
